# baseline (speedup 1.0000x reference)
import jax
import jax.numpy as jnp
from jax import lax
from jax.experimental import pallas as pl
from jax.experimental.pallas import tpu as pltpu
from jax.experimental.pallas import tpu_sc as plsc

BATCH = 16384
EMB = 4
VOCAB_R = 14604
NC = 2
NS = 16
NW = NC * NS
BPW = BATCH // NW
HPW = BPW * EMB


def _sc_gather_body(embr_hbm, x0_hbm, x1_hbm, h_hbm,
                    idx0_v, idx1_v, sr0_v, sr1_v, rows0_v, rows1_v, h_v,
                    sem0, sem1):
    wid = lax.axis_index("s") * NC + lax.axis_index("c")
    base = wid * BPW
    pltpu.sync_copy(x0_hbm.at[pl.ds(base, BPW)], idx0_v)
    pltpu.sync_copy(x1_hbm.at[pl.ds(base, BPW)], idx1_v)
    for c in range(BPW // 16):
        s = pl.ds(16 * c, 16)
        sr0_v[s] = idx0_v[s] >> 2
        sr1_v[s] = idx1_v[s] >> 2
    c0 = pltpu.async_copy(embr_hbm.at[sr0_v], rows0_v, sem0)
    c1 = pltpu.async_copy(embr_hbm.at[sr1_v], rows1_v, sem1)
    c0.wait()
    c1.wait()
    iota = lax.iota(jnp.int32, 16)
    brow = iota >> 2
    lcol = iota & 3
    for t in range(BPW // 4):
        row = brow + 4 * t
        s0 = plsc.load_gather(idx0_v, [row]) & 3
        s1 = plsc.load_gather(idx1_v, [row]) & 3
        v0 = plsc.load_gather(rows0_v, [row, s0 * 4 + lcol])
        v1 = plsc.load_gather(rows1_v, [row, s1 * 4 + lcol])
        h_v[pl.ds(16 * t, 16)] = v0 * v1
    pltpu.sync_copy(h_v, h_hbm.at[pl.ds(wid * HPW, HPW)])


def _sc_gather(embr, x0, x1):
    mesh = plsc.VectorSubcoreMesh(core_axis_name="c", subcore_axis_name="s")
    fn = pl.kernel(
        _sc_gather_body,
        mesh=mesh,
        out_type=jax.ShapeDtypeStruct((BATCH * EMB,), jnp.float32),
        scratch_types=[
            pltpu.VMEM((BPW,), jnp.int32),
            pltpu.VMEM((BPW,), jnp.int32),
            pltpu.VMEM((BPW,), jnp.int32),
            pltpu.VMEM((BPW,), jnp.int32),
            pltpu.VMEM((BPW, 16), jnp.float32),
            pltpu.VMEM((BPW, 16), jnp.float32),
            pltpu.VMEM((HPW,), jnp.float32),
            pltpu.SemaphoreType.DMA,
            pltpu.SemaphoreType.DMA,
        ],
        compiler_params=pltpu.CompilerParams(
            use_tc_tiling_on_sc=False, needs_layout_passes=False),
    )
    return fn(embr, x0, x1)


BLK = 2048


def _mlp_body(h_ref, w1_ref, b1_ref, w2_ref, b2_ref, w3_ref, b3_ref,
              out_ref):
    dn = (((1,), (0,)), ((), ()))
    h1 = lax.dot_general(h_ref[...], w1_ref[...], dn,
                         preferred_element_type=jnp.float32)
    h1 = jnp.maximum(h1 + b1_ref[...], 0.0)
    h2 = lax.dot_general(h1, w2_ref[...], dn,
                         preferred_element_type=jnp.float32)
    h2 = jnp.maximum(h2 + b2_ref[...], 0.0)
    out = lax.dot_general(h2, w3_ref[...], dn,
                          preferred_element_type=jnp.float32)
    out_ref[...] = out + b3_ref[...]


def _tc_mlp(h, W1, b1, W2, b2, W3, b3):
    grid = (BATCH // BLK,)
    full = lambda shape: pl.BlockSpec(shape, lambda i: (0, 0))
    return pl.pallas_call(
        _mlp_body,
        grid=grid,
        in_specs=[
            pl.BlockSpec((BLK, EMB), lambda i: (i, 0)),
            full(W1.shape),
            full((1, 64)),
            full(W2.shape),
            full((1, 32)),
            full(W3.shape),
            full((1, 2)),
        ],
        out_specs=pl.BlockSpec((BLK, 2), lambda i: (i, 0)),
        out_shape=jax.ShapeDtypeStruct((BATCH, 2), jnp.float32),
    )(h, W1, b1, W2, b2, W3, b3)


@jax.jit
def kernel(x, emb, W1, b1, W2, b2, W3, b3):
    x0 = x[:, 0].astype(jnp.int32)
    x1 = x[:, 1].astype(jnp.int32)
    embr = emb.reshape(VOCAB_R, 4 * EMB)
    h = _sc_gather(embr, x0, x1).reshape(BATCH, EMB)
    return _tc_mlp(h, W1, b1.reshape(1, -1), W2, b2.reshape(1, -1),
                   W3, b3.reshape(1, -1))
